# baseline (speedup 1.0000x reference)
import functools

import jax
import jax.numpy as jnp
from jax import lax
from jax.experimental import pallas as pl
from jax.experimental.pallas import tpu as pltpu, tpu_sc as plsc

N = 100000
D = 128
K = 25000

NC = 2
NS = 16
NW = NC * NS

K_PAD = 25088
HALF = K_PAD // 2
B_PER_W = HALF // NW
K16 = K_PAD // NS

NCHUNK = 10
CPS = NCHUNK // NC
C = N // NCHUNK
CT = 624
CT_LAST = C - (NS - 1) * CT
LCAP = K16 + 32

_mesh = plsc.VectorSubcoreMesh(core_axis_name="c", subcore_axis_name="s")
_sc_params = pltpu.CompilerParams(needs_layout_passes=False)


@functools.partial(
    pl.kernel,
    out_type=jax.ShapeDtypeStruct((HALF, D), jnp.float32),
    mesh=_mesh,
    scratch_types=[
        pltpu.VMEM((B_PER_W,), jnp.int32),
        pltpu.VMEM((B_PER_W, D), jnp.float32),
        pltpu.SemaphoreType.DMA,
    ],
    compiler_params=_sc_params,
)
def _sc_gather(x_hbm, idx_hbm, out_hbm, idx_v, rows_v, sem):
    wid = lax.axis_index("s") * NC + lax.axis_index("c")
    base = wid * B_PER_W
    pltpu.sync_copy(idx_hbm.at[pl.ds(base, B_PER_W)], idx_v)
    pltpu.async_copy(x_hbm.at[idx_v], rows_v, sem).wait()
    pltpu.sync_copy(rows_v, out_hbm.at[pl.ds(base, B_PER_W)])


def _dense_body(xg_ref, w_ref, b_ref, o_ref):
    h = jnp.dot(xg_ref[...], w_ref[...], preferred_element_type=jnp.float32)
    h = h + b_ref[...]
    o_ref[...] = jnp.where(h > 0.0, h, jnp.exp(jnp.minimum(h, 0.0)) - 1.0)


def _tc_dense(up, w0, b0):
    blk = 12544
    grid = (HALF // blk,)
    return pl.pallas_call(
        _dense_body,
        grid=grid,
        in_specs=[
            pl.BlockSpec((blk, D), lambda i: (i, 0)),
            pl.BlockSpec((D, D), lambda i: (0, 0)),
            pl.BlockSpec((1, D), lambda i: (0, 0)),
        ],
        out_specs=pl.BlockSpec((blk, D), lambda i: (i, 0)),
        out_shape=jax.ShapeDtypeStruct((HALF, D), jnp.float32),
    )(up, w0, b0.reshape(1, D))


@functools.partial(
    pl.kernel,
    out_type=jax.ShapeDtypeStruct((N, D), jnp.float32),
    mesh=_mesh,
    scratch_types=[
        pltpu.VMEM_SHARED((C + 16, D), jnp.float32),
        pltpu.VMEM((K16,), jnp.int32),
        pltpu.VMEM((LCAP,), jnp.int32),
        pltpu.VMEM((LCAP,), jnp.int32),
        pltpu.VMEM((16, D), jnp.float32),
        pltpu.VMEM((16, D), jnp.float32),
        pltpu.SemaphoreType.DMA,
        pltpu.SemaphoreType.DMA,
    ],
    compiler_params=_sc_params,
)
def _sc_scatter(x_hbm, idx_hbm, y0_hbm, y1_hbm, out_hbm,
                acc_sh, idx_v, jlist_v, llist_v, rows_a, rows_b,
                sem_a, sem_b):
    cid = lax.axis_index("c")
    sid = lax.axis_index("s")

    pltpu.sync_copy(idx_hbm.at[pl.ds(sid * K16, K16)], idx_v)
    lane = lax.iota(jnp.int32, 16)

    for k in range(CPS):
        lo = (cid * CPS + k) * C

        @pl.when(sid < NS - 1)
        def _():
            pltpu.sync_copy(x_hbm.at[pl.ds(lo + sid * CT, CT)],
                            acc_sh.at[pl.ds(sid * CT, CT)])

        @pl.when(sid == NS - 1)
        def _():
            pltpu.sync_copy(x_hbm.at[pl.ds(lo + (NS - 1) * CT, CT_LAST)],
                            acc_sh.at[pl.ds((NS - 1) * CT, CT_LAST)])

        def scan_body(v, cnt):
            ivec = idx_v[pl.ds(v * 16, 16)]
            pos = sid * K16 + v * 16 + lane
            m = (ivec >= lo) & (ivec < lo + C) & (pos < K)
            mi = jnp.where(m, jnp.int32(1), jnp.int32(0))
            csum = plsc.cumsum(mi)
            slots = cnt + csum - 1
            plsc.store_scatter(llist_v, [slots], ivec - lo, mask=m)
            plsc.store_scatter(jlist_v, [slots], pos, mask=m)
            return cnt + csum[15]

        cnt = lax.fori_loop(0, K16 // 16, scan_body, jnp.int32(0))

        for p in range(2):
            llist_v[pl.ds(cnt + p * 16, 16)] = jnp.full((16,), C, jnp.int32)
            jlist_v[pl.ds(cnt + p * 16, 16)] = lane * 0 + sid * K16

        plsc.subcore_barrier()

        nb = (cnt + 15) // 16

        def mk_batch(yref, joff):
            def batch_body(b, carry):
                jv = jlist_v[pl.ds(b * 16, 16)] - joff
                pltpu.async_copy(yref.at[jv], rows_a, sem_a).wait()
                lv = llist_v[pl.ds(b * 16, 16)]
                pltpu.sync_copy(rows_a, acc_sh.at[lv], add=True)
                return carry
            return batch_body

        @pl.when(sid < NS // 2)
        def _():
            lax.fori_loop(0, nb, mk_batch(y0_hbm, 0), jnp.int32(0))

        @pl.when(sid >= NS // 2)
        def _():
            lax.fori_loop(0, nb, mk_batch(y1_hbm, HALF), jnp.int32(0))

        plsc.subcore_barrier()

        @pl.when(sid < NS - 1)
        def _():
            pltpu.sync_copy(acc_sh.at[pl.ds(sid * CT, CT)],
                            out_hbm.at[pl.ds(lo + sid * CT, CT)])

        @pl.when(sid == NS - 1)
        def _():
            pltpu.sync_copy(acc_sh.at[pl.ds((NS - 1) * CT, CT_LAST)],
                            out_hbm.at[pl.ds(lo + (NS - 1) * CT, CT_LAST)])

        plsc.subcore_barrier()


def kernel(x, sel_idx_up, W0, b0):
    idx = sel_idx_up[:, 0]
    idx_pad = jnp.concatenate(
        [idx, jnp.zeros((K_PAD - K,), dtype=jnp.int32)])
    up0 = _sc_gather(x, idx_pad[:HALF])
    up1 = _sc_gather(x, idx_pad[HALF:])
    y0 = _tc_dense(up0, W0, b0)
    y1 = _tc_dense(up1, W0, b0)
    return _sc_scatter(x, idx_pad, y0, y1)


# speedup vs baseline: 1.0183x; 1.0183x over previous
import functools

import jax
import jax.numpy as jnp
from jax import lax
from jax.experimental import pallas as pl
from jax.experimental.pallas import tpu as pltpu, tpu_sc as plsc

N = 100000
D = 128
K = 25000

NC = 2
NS = 16
NW = NC * NS

K_PAD = 25088
HALF = K_PAD // 2
B_PER_W = HALF // NW
K16 = K_PAD // NS

NCHUNK = 10
CPS = NCHUNK // NC
C = N // NCHUNK
CT = 624
CT_LAST = C - (NS - 1) * CT
LCAP = K16 + 32

_mesh = plsc.VectorSubcoreMesh(core_axis_name="c", subcore_axis_name="s")
_sc_params = pltpu.CompilerParams(needs_layout_passes=False)


@functools.partial(
    pl.kernel,
    out_type=jax.ShapeDtypeStruct((HALF, D), jnp.float32),
    mesh=_mesh,
    scratch_types=[
        pltpu.VMEM((B_PER_W,), jnp.int32),
        pltpu.VMEM((B_PER_W, D), jnp.float32),
        pltpu.SemaphoreType.DMA,
    ],
    compiler_params=_sc_params,
)
def _sc_gather(x_hbm, idx_hbm, out_hbm, idx_v, rows_v, sem):
    wid = lax.axis_index("s") * NC + lax.axis_index("c")
    base = wid * B_PER_W
    pltpu.sync_copy(idx_hbm.at[pl.ds(base, B_PER_W)], idx_v)
    pltpu.async_copy(x_hbm.at[idx_v], rows_v, sem).wait()
    pltpu.sync_copy(rows_v, out_hbm.at[pl.ds(base, B_PER_W)])


def _dense_body(xg_ref, w_ref, b_ref, o_ref):
    h = jnp.dot(xg_ref[...], w_ref[...], preferred_element_type=jnp.float32)
    h = h + b_ref[...]
    o_ref[...] = jnp.where(h > 0.0, h, jnp.exp(jnp.minimum(h, 0.0)) - 1.0)


def _tc_dense(up, w0, b0):
    blk = 6272
    grid = (HALF // blk,)
    return pl.pallas_call(
        _dense_body,
        grid=grid,
        in_specs=[
            pl.BlockSpec((blk, D), lambda i: (i, 0)),
            pl.BlockSpec((D, D), lambda i: (0, 0)),
            pl.BlockSpec((1, D), lambda i: (0, 0)),
        ],
        out_specs=pl.BlockSpec((blk, D), lambda i: (i, 0)),
        out_shape=jax.ShapeDtypeStruct((HALF, D), jnp.float32),
    )(up, w0, b0.reshape(1, D))


@functools.partial(
    pl.kernel,
    out_type=jax.ShapeDtypeStruct((N, D), jnp.float32),
    mesh=_mesh,
    scratch_types=[
        pltpu.VMEM_SHARED((C + 16, D), jnp.float32),
        pltpu.VMEM((K16,), jnp.int32),
        pltpu.VMEM((LCAP,), jnp.int32),
        pltpu.VMEM((LCAP,), jnp.int32),
        pltpu.VMEM((16, D), jnp.float32),
        pltpu.VMEM((16, D), jnp.float32),
        pltpu.SemaphoreType.DMA,
        pltpu.SemaphoreType.DMA,
    ],
    compiler_params=_sc_params,
)
def _sc_scatter(x_hbm, idx_hbm, y0_hbm, y1_hbm, out_hbm,
                acc_sh, idx_v, jlist_v, llist_v, rows_a, rows_b,
                sem_a, sem_b):
    cid = lax.axis_index("c")
    sid = lax.axis_index("s")

    pltpu.sync_copy(idx_hbm.at[pl.ds(sid * K16, K16)], idx_v)
    lane = lax.iota(jnp.int32, 16)

    for k in range(CPS):
        lo = (cid * CPS + k) * C

        @pl.when(sid < NS - 1)
        def _():
            pltpu.sync_copy(x_hbm.at[pl.ds(lo + sid * CT, CT)],
                            acc_sh.at[pl.ds(sid * CT, CT)])

        @pl.when(sid == NS - 1)
        def _():
            pltpu.sync_copy(x_hbm.at[pl.ds(lo + (NS - 1) * CT, CT_LAST)],
                            acc_sh.at[pl.ds((NS - 1) * CT, CT_LAST)])

        def scan_body(v, cnt):
            ivec = idx_v[pl.ds(v * 16, 16)]
            pos = sid * K16 + v * 16 + lane
            m = (ivec >= lo) & (ivec < lo + C) & (pos < K)
            mi = jnp.where(m, jnp.int32(1), jnp.int32(0))
            csum = plsc.cumsum(mi)
            slots = cnt + csum - 1
            plsc.store_scatter(llist_v, [slots], ivec - lo, mask=m)
            plsc.store_scatter(jlist_v, [slots], pos, mask=m)
            return cnt + csum[15]

        cnt = lax.fori_loop(0, K16 // 16, scan_body, jnp.int32(0))

        for p in range(2):
            llist_v[pl.ds(cnt + p * 16, 16)] = jnp.full((16,), C, jnp.int32)
            jlist_v[pl.ds(cnt + p * 16, 16)] = lane * 0 + sid * K16

        plsc.subcore_barrier()

        nb = (cnt + 15) // 16

        def mk_batch(yref, joff):
            def batch_body(b, carry):
                jv = jlist_v[pl.ds(b * 16, 16)] - joff
                pltpu.async_copy(yref.at[jv], rows_a, sem_a).wait()
                lv = llist_v[pl.ds(b * 16, 16)]
                pltpu.sync_copy(rows_a, acc_sh.at[lv], add=True)
                return carry
            return batch_body

        @pl.when(sid < NS // 2)
        def _():
            lax.fori_loop(0, nb, mk_batch(y0_hbm, 0), jnp.int32(0))

        @pl.when(sid >= NS // 2)
        def _():
            lax.fori_loop(0, nb, mk_batch(y1_hbm, HALF), jnp.int32(0))

        plsc.subcore_barrier()

        @pl.when(sid < NS - 1)
        def _():
            pltpu.sync_copy(acc_sh.at[pl.ds(sid * CT, CT)],
                            out_hbm.at[pl.ds(lo + sid * CT, CT)])

        @pl.when(sid == NS - 1)
        def _():
            pltpu.sync_copy(acc_sh.at[pl.ds((NS - 1) * CT, CT_LAST)],
                            out_hbm.at[pl.ds(lo + (NS - 1) * CT, CT_LAST)])

        plsc.subcore_barrier()


def kernel(x, sel_idx_up, W0, b0):
    idx = sel_idx_up[:, 0]
    idx_pad = jnp.concatenate(
        [idx, jnp.zeros((K_PAD - K,), dtype=jnp.int32)])
    up0 = _sc_gather(x, idx_pad[:HALF])
    up1 = _sc_gather(x, idx_pad[HALF:])
    y0 = _tc_dense(up0, W0, b0)
    y1 = _tc_dense(up1, W0, b0)
    return _sc_scatter(x, idx_pad, y0, y1)
